# baseline (device time: 36191 ns/iter reference)
import jax
import jax.numpy as jnp
from jax import lax
from jax.experimental import pallas as pl
from jax.experimental.pallas import tpu as pltpu

N_DEV = 4
N_TOK = 2048
D = 512
H = 1024
E_LOCAL = 8
CHUNK = N_TOK // N_DEV
CAP = 192


def kernel(x, router_W, route_idx, expert_W, shared_W):
    ridx_lane = route_idx.astype(jnp.float32).T

    def body(x_ref, rw_ref, idx_ref, rl_ref, ew_ref, sw_ref, out_ref,
             send_ref, recv_ref, xbf_ref, send_sems, recv_sems):
        me = lax.axis_index("i")
        lo = me * E_LOCAL
        lo_f = lo.astype(jnp.float32)

        barrier_sem = pltpu.get_barrier_semaphore()
        for k in range(1, N_DEV):
            pl.semaphore_signal(
                barrier_sem, inc=1,
                device_id=(lax.rem(me + k, N_DEV),),
                device_id_type=pl.DeviceIdType.MESH,
            )
        pl.semaphore_wait(barrier_sem, N_DEV - 1)

        t_row = lax.broadcasted_iota(jnp.int32, (CHUNK, CHUNK), 0)
        t_col = lax.broadcasted_iota(jnp.int32, (CHUNK, CHUNK), 1)
        upper = jnp.where(t_row < t_col, 1.0, 0.0).astype(jnp.float32)
        lower = jnp.where(t_col < t_row, 1.0, 0.0).astype(jnp.float32)

        xbf_ref[...] = x_ref[...].astype(jnp.bfloat16)
        ew_bf = ew_ref[...].astype(jnp.bfloat16)
        sw_bf = sw_ref[...].astype(jnp.bfloat16)

        def compact_contrib(d):
            row0 = d * CHUNK
            x_d = x_ref[pl.ds(row0, CHUNK), :]
            ridx_d = idx_ref[pl.ds(row0, CHUNK), :]

            scores = jnp.dot(x_d, rw_ref[:, :],
                             preferred_element_type=jnp.float32)
            scores = scores - jnp.max(scores, axis=1, keepdims=True)
            p = jnp.exp(scores)
            p = p / jnp.sum(p, axis=1, keepdims=True)
            cols = lax.broadcasted_iota(jnp.int32, scores.shape, 1)
            gate = jnp.sum(jnp.where(cols == ridx_d, p, 0.0),
                           axis=1, keepdims=True)

            rl = rl_ref[:, pl.ds(row0, CHUNK)]
            mine_l = jnp.where(
                (rl >= lo_f) & (rl < lo_f + E_LOCAL), 1.0, 0.0)
            rank_l = jnp.dot(mine_l, upper,
                             preferred_element_type=jnp.float32)
            r_iota = lax.broadcasted_iota(jnp.int32, (CAP, CHUNK), 0).astype(jnp.float32)
            G = jnp.where((r_iota == rank_l) & (mine_l > 0.0), 1.0, 0.0)

            G_bf = G.astype(jnp.bfloat16)
            xm = jnp.dot(G_bf, xbf_ref[pl.ds(row0, CHUNK), :],
                         preferred_element_type=jnp.float32)
            gcomp = jnp.dot(G, gate, preferred_element_type=jnp.float32)
            rcomp = jnp.dot(G, ridx_d.astype(jnp.float32),
                            preferred_element_type=jnp.float32)
            xm_bf = (xm * gcomp).astype(jnp.bfloat16)

            acc = jnp.zeros((CAP, H), dtype=jnp.float32)
            for k in range(E_LOCAL):
                e_k = lo_f + k
                sel = jnp.where(rcomp == e_k, 1.0, 0.0).astype(jnp.bfloat16)
                acc = acc + jnp.dot(xm_bf * sel, ew_bf[k, :, :],
                                    preferred_element_type=jnp.float32)
            return acc

        def scatter_onehot(src):
            ridx_d = idx_ref[pl.ds(me * CHUNK, CHUNK), :]
            slo = src * E_LOCAL
            mine_s = jnp.where((ridx_d >= slo) & (ridx_d < slo + E_LOCAL),
                               1.0, 0.0)
            rank_s = jnp.dot(lower, mine_s,
                             preferred_element_type=jnp.float32)
            c_iota = lax.broadcasted_iota(jnp.int32, (CHUNK, CAP), 1).astype(jnp.float32)
            return jnp.where((c_iota == rank_s) & (mine_s > 0.0), 1.0, 0.0)

        rdmas = []
        for k in (2, 1, 3):
            d = lax.rem(me + k, N_DEV)
            send_ref[k - 1, :, :] = compact_contrib(d).astype(jnp.bfloat16)
            rdma = pltpu.make_async_remote_copy(
                src_ref=send_ref.at[k - 1],
                dst_ref=recv_ref.at[3 - k],
                send_sem=send_sems.at[k - 1],
                recv_sem=recv_sems.at[3 - k],
                device_id=(d,),
                device_id_type=pl.DeviceIdType.MESH,
            )
            rdma.start()
            rdmas.append(rdma)

        acc_me = compact_contrib(me)
        out_ref[...] = (
            jnp.dot(scatter_onehot(me).astype(jnp.bfloat16),
                    acc_me.astype(jnp.bfloat16),
                    preferred_element_type=jnp.float32)
            + jnp.dot(xbf_ref[pl.ds(me * CHUNK, CHUNK), :], sw_bf,
                      preferred_element_type=jnp.float32)
        )

        for j in (1, 2, 0):
            src = lax.rem(me + j + 1, N_DEV)
            recv = pltpu.make_async_remote_copy(
                src_ref=send_ref.at[0],
                dst_ref=recv_ref.at[j],
                send_sem=send_sems.at[0],
                recv_sem=recv_sems.at[j],
                device_id=(src,),
                device_id_type=pl.DeviceIdType.MESH,
            )
            recv.wait_recv()
            out_ref[...] = out_ref[...] + jnp.dot(
                scatter_onehot(src).astype(jnp.bfloat16), recv_ref[j],
                preferred_element_type=jnp.float32)

        for rdma in rdmas:
            rdma.wait_send()

    return pl.pallas_call(
        body,
        out_shape=jax.ShapeDtypeStruct((CHUNK, H), jnp.float32),
        in_specs=[
            pl.BlockSpec(memory_space=pltpu.VMEM),
            pl.BlockSpec(memory_space=pltpu.VMEM),
            pl.BlockSpec(memory_space=pltpu.VMEM),
            pl.BlockSpec(memory_space=pltpu.VMEM),
            pl.BlockSpec(memory_space=pltpu.VMEM),
            pl.BlockSpec(memory_space=pltpu.VMEM),
        ],
        out_specs=pl.BlockSpec(memory_space=pltpu.VMEM),
        scratch_shapes=[
            pltpu.VMEM((N_DEV - 1, CAP, H), jnp.bfloat16),
            pltpu.VMEM((N_DEV - 1, CAP, H), jnp.bfloat16),
            pltpu.VMEM((N_TOK, D), jnp.bfloat16),
            pltpu.SemaphoreType.DMA((N_DEV - 1,)),
            pltpu.SemaphoreType.DMA((N_DEV - 1,)),
        ],
        compiler_params=pltpu.CompilerParams(
            collective_id=0, vmem_limit_bytes=96 * 1024 * 1024),
    )(x, router_W, route_idx, ridx_lane, expert_W, shared_W)


# device time: 32106 ns/iter; 1.1272x vs baseline; 1.1272x over previous
import jax
import jax.numpy as jnp
from jax import lax
from jax.experimental import pallas as pl
from jax.experimental.pallas import tpu as pltpu

N_DEV = 4
N_TOK = 2048
D = 512
H = 1024
E_LOCAL = 8
CHUNK = N_TOK // N_DEV
CAP = 192


def kernel(x, router_W, route_idx, expert_W, shared_W):
    ridx_lane = route_idx.astype(jnp.float32).T

    def body(x_ref, rw_ref, idx_ref, rl_ref, ew_ref, sw_ref, out_ref,
             send_ref, recv_ref, send_sems, recv_sems):
        me = lax.axis_index("i")
        lo = me * E_LOCAL
        lo_f = lo.astype(jnp.float32)

        barrier_sem = pltpu.get_barrier_semaphore()
        for k in range(1, N_DEV):
            pl.semaphore_signal(
                barrier_sem, inc=1,
                device_id=(lax.rem(me + k, N_DEV),),
                device_id_type=pl.DeviceIdType.MESH,
            )
        pl.semaphore_wait(barrier_sem, N_DEV - 1)

        t_row = lax.broadcasted_iota(jnp.int32, (CHUNK, CHUNK), 0)
        t_col = lax.broadcasted_iota(jnp.int32, (CHUNK, CHUNK), 1)
        upper = jnp.where(t_row < t_col, 1.0, 0.0).astype(jnp.float32)
        lower = jnp.where(t_col < t_row, 1.0, 0.0).astype(jnp.float32)
        ew_flat = jnp.reshape(ew_ref[...], (E_LOCAL * D, H))

        def compact_contrib(d):
            row0 = d * CHUNK
            x_d = x_ref[pl.ds(row0, CHUNK), :]
            ridx_d = idx_ref[pl.ds(row0, CHUNK), :]

            scores = jnp.dot(x_d, rw_ref[:, :],
                             preferred_element_type=jnp.float32)
            scores = scores - jnp.max(scores, axis=1, keepdims=True)
            p = jnp.exp(scores)
            p = p / jnp.sum(p, axis=1, keepdims=True)
            cols = lax.broadcasted_iota(jnp.int32, scores.shape, 1)
            gate = jnp.sum(jnp.where(cols == ridx_d, p, 0.0),
                           axis=1, keepdims=True)

            rl = rl_ref[:, pl.ds(row0, CHUNK)]
            mine_l = jnp.where(
                (rl >= lo_f) & (rl < lo_f + E_LOCAL), 1.0, 0.0)
            rank_l = jnp.dot(mine_l, upper,
                             preferred_element_type=jnp.float32)
            r_iota = lax.broadcasted_iota(jnp.int32, (CAP, CHUNK), 0).astype(jnp.float32)
            G = jnp.where((r_iota == rank_l) & (mine_l > 0.0), 1.0, 0.0)

            xm = jnp.dot(G, x_d, preferred_element_type=jnp.float32)
            gcomp = jnp.dot(G, gate, preferred_element_type=jnp.float32)
            rcomp = jnp.dot(G, ridx_d.astype(jnp.float32),
                            preferred_element_type=jnp.float32)

            xcat = jnp.concatenate(
                [xm * jnp.where(rcomp == lo_f + k, gcomp, 0.0)
                 for k in range(E_LOCAL)], axis=1)
            return jnp.dot(xcat, ew_flat,
                           preferred_element_type=jnp.float32)

        def scatter_onehot(src):
            ridx_d = idx_ref[pl.ds(me * CHUNK, CHUNK), :]
            slo = src * E_LOCAL
            mine_s = jnp.where((ridx_d >= slo) & (ridx_d < slo + E_LOCAL),
                               1.0, 0.0)
            rank_s = jnp.dot(lower, mine_s,
                             preferred_element_type=jnp.float32)
            c_iota = lax.broadcasted_iota(jnp.int32, (CHUNK, CAP), 1).astype(jnp.float32)
            return jnp.where((c_iota == rank_s) & (mine_s > 0.0), 1.0, 0.0)

        rdmas = []
        for k in (2, 1, 3):
            d = lax.rem(me + k, N_DEV)
            send_ref[k - 1, :, :] = compact_contrib(d).astype(jnp.bfloat16)
            rdma = pltpu.make_async_remote_copy(
                src_ref=send_ref.at[k - 1],
                dst_ref=recv_ref.at[3 - k],
                send_sem=send_sems.at[k - 1],
                recv_sem=recv_sems.at[3 - k],
                device_id=(d,),
                device_id_type=pl.DeviceIdType.MESH,
            )
            rdma.start()
            rdmas.append(rdma)

        acc_me = compact_contrib(me)
        x_me = x_ref[pl.ds(me * CHUNK, CHUNK), :]
        out_ref[...] = (
            jnp.dot(scatter_onehot(me), acc_me,
                    preferred_element_type=jnp.float32)
            + jnp.dot(x_me, sw_ref[:, :], preferred_element_type=jnp.float32)
        )

        for j in (1, 2, 0):
            src = lax.rem(me + j + 1, N_DEV)
            recv = pltpu.make_async_remote_copy(
                src_ref=send_ref.at[0],
                dst_ref=recv_ref.at[j],
                send_sem=send_sems.at[0],
                recv_sem=recv_sems.at[j],
                device_id=(src,),
                device_id_type=pl.DeviceIdType.MESH,
            )
            recv.wait_recv()
            out_ref[...] = out_ref[...] + jnp.dot(
                scatter_onehot(src).astype(jnp.bfloat16), recv_ref[j],
                preferred_element_type=jnp.float32)

        for rdma in rdmas:
            rdma.wait_send()

    return pl.pallas_call(
        body,
        out_shape=jax.ShapeDtypeStruct((CHUNK, H), jnp.float32),
        in_specs=[
            pl.BlockSpec(memory_space=pltpu.VMEM),
            pl.BlockSpec(memory_space=pltpu.VMEM),
            pl.BlockSpec(memory_space=pltpu.VMEM),
            pl.BlockSpec(memory_space=pltpu.VMEM),
            pl.BlockSpec(memory_space=pltpu.VMEM),
            pl.BlockSpec(memory_space=pltpu.VMEM),
        ],
        out_specs=pl.BlockSpec(memory_space=pltpu.VMEM),
        scratch_shapes=[
            pltpu.VMEM((N_DEV - 1, CAP, H), jnp.bfloat16),
            pltpu.VMEM((N_DEV - 1, CAP, H), jnp.bfloat16),
            pltpu.SemaphoreType.DMA((N_DEV - 1,)),
            pltpu.SemaphoreType.DMA((N_DEV - 1,)),
        ],
        compiler_params=pltpu.CompilerParams(collective_id=0),
    )(x, router_W, route_idx, ridx_lane, expert_W, shared_W)
